# baseline (speedup 1.0000x reference)
import functools

import jax
import jax.numpy as jnp
from jax import lax
from jax.experimental import pallas as pl
from jax.experimental.pallas import tpu as pltpu
from jax.experimental.pallas import tpu_sc as plsc

N_NODES = 100000
N_PAD = 102400
N_EDGES = 1600000
F = 32

_NODE_BLK = 2000
_EDGE_BLK = 12800

_NC = 2
_NS = 16
_NW = _NC * _NS
_EPT = N_EDGES // _NW
_GCH = 1000
_GNCH = _EPT // _GCH


def _gather_diff_relu(A, B, src, dst):
    mesh = plsc.VectorSubcoreMesh(core_axis_name="c", subcore_axis_name="s")

    @functools.partial(
        pl.kernel, mesh=mesh,
        out_type=jax.ShapeDtypeStruct((N_EDGES, F), jnp.float32),
        compiler_params=pltpu.CompilerParams(use_tc_tiling_on_sc=False),
        scratch_types=[
            pltpu.VMEM((_GCH,), jnp.int32),
            pltpu.VMEM((_GCH,), jnp.int32),
            pltpu.VMEM((_GCH, F), jnp.float32),
            pltpu.VMEM((_GCH, F), jnp.float32),
            pltpu.SemaphoreType.DMA,
        ],
    )
    def k(a_hbm, b_hbm, src_hbm, dst_hbm, out_hbm, si_v, di_v, ar_v, br_v, sem):
        wid = lax.axis_index("s") * _NC + lax.axis_index("c")
        base = wid * _EPT

        def chunk_body(ci, carry):
            off = base + ci * _GCH
            pltpu.sync_copy(src_hbm.at[pl.ds(off, _GCH)], si_v)
            pltpu.sync_copy(dst_hbm.at[pl.ds(off, _GCH)], di_v)
            pltpu.async_copy(a_hbm.at[si_v], ar_v, sem).wait()
            pltpu.async_copy(b_hbm.at[di_v], br_v, sem).wait()

            def row_body(i, c2):
                for j in range(F // 16):
                    sl = pl.ds(j * 16, 16)
                    ar_v[i, sl] = jnp.maximum(ar_v[i, sl] - br_v[i, sl], 0.0)
                return c2

            lax.fori_loop(0, _GCH, row_body, 0)
            pltpu.sync_copy(ar_v, out_hbm.at[pl.ds(off, _GCH)])
            return carry

        lax.fori_loop(0, _GNCH, chunk_body, 0)

    return k(A, B, src, dst)


def _node_tables_body(h_ref, pos_ref, wh_ref, wp_ref, b_ref, a_ref, b_out_ref):
    pq = jax.lax.dot_general(pos_ref[...], wp_ref[...], (((1,), (0,)), ((), ())),
                             preferred_element_type=jnp.float32)
    hq = jax.lax.dot_general(h_ref[...], wh_ref[...], (((1,), (0,)), ((), ())),
                             preferred_element_type=jnp.float32)
    a_ref[...] = hq + pq + b_ref[...]
    b_out_ref[...] = pq


def _node_tables(h, pos, Wh, Wp, b):
    n = h.shape[0]
    k = h.shape[1]
    grid = n // _NODE_BLK
    return pl.pallas_call(
        _node_tables_body,
        grid=(grid,),
        in_specs=[
            pl.BlockSpec((_NODE_BLK, k), lambda i: (i, 0)),
            pl.BlockSpec((_NODE_BLK, 3), lambda i: (i, 0)),
            pl.BlockSpec((k, F), lambda i: (0, 0)),
            pl.BlockSpec((3, F), lambda i: (0, 0)),
            pl.BlockSpec((1, F), lambda i: (0, 0)),
        ],
        out_specs=[
            pl.BlockSpec((_NODE_BLK, F), lambda i: (i, 0)),
            pl.BlockSpec((_NODE_BLK, F), lambda i: (i, 0)),
        ],
        out_shape=[
            jax.ShapeDtypeStruct((n, F), jnp.float32),
            jax.ShapeDtypeStruct((n, F), jnp.float32),
        ],
    )(h, pos, Wh, Wp, b[None, :])


def _edge_mlp_t_body(e_ref, wb_ref, bb_ref, out_ref):
    out_ref[...] = jax.lax.dot_general(
        wb_ref[...], e_ref[...], (((0,), (1,)), ((), ())),
        preferred_element_type=jnp.float32) + bb_ref[...]


def _edge_mlp_t(e, Wb, bb):
    ne = e.shape[0]
    grid = ne // _EDGE_BLK
    return pl.pallas_call(
        _edge_mlp_t_body,
        grid=(grid,),
        in_specs=[
            pl.BlockSpec((_EDGE_BLK, F), lambda i: (i, 0)),
            pl.BlockSpec((F, F), lambda i: (0, 0)),
            pl.BlockSpec((F, 1), lambda i: (0, 0)),
        ],
        out_specs=pl.BlockSpec((F, _EDGE_BLK), lambda i: (0, i)),
        out_shape=jax.ShapeDtypeStruct((F, ne), jnp.float32),
    )(e, Wb, bb[:, None])


def _node_tables_t_body(ht_ref, pos_ref, wh_ref, wp_ref, b_ref, a_ref, b_out_ref):
    pq = jax.lax.dot_general(pos_ref[...], wp_ref[...], (((1,), (0,)), ((), ())),
                             preferred_element_type=jnp.float32)
    hq = jax.lax.dot_general(ht_ref[...], wh_ref[...], (((0,), (0,)), ((), ())),
                             preferred_element_type=jnp.float32)
    a_ref[...] = hq + pq + b_ref[...]
    b_out_ref[...] = pq


def _node_tables_t(ht, pos, Wh, Wp, b):
    n = pos.shape[0]
    blk = 12800
    grid = n // blk
    return pl.pallas_call(
        _node_tables_t_body,
        grid=(grid,),
        in_specs=[
            pl.BlockSpec((F, blk), lambda i: (0, i)),
            pl.BlockSpec((blk, 3), lambda i: (i, 0)),
            pl.BlockSpec((F, F), lambda i: (0, 0)),
            pl.BlockSpec((3, F), lambda i: (0, 0)),
            pl.BlockSpec((1, F), lambda i: (0, 0)),
        ],
        out_specs=[
            pl.BlockSpec((blk, F), lambda i: (i, 0)),
            pl.BlockSpec((blk, F), lambda i: (i, 0)),
        ],
        out_shape=[
            jax.ShapeDtypeStruct((n, F), jnp.float32),
            jax.ShapeDtypeStruct((n, F), jnp.float32),
        ],
    )(ht, pos, Wh, Wp, b[None, :])


_SCH = 8000
_SNCH = N_EDGES // _SCH


def _scatter_max(mt, dst):
    mesh = plsc.VectorSubcoreMesh(core_axis_name="c", subcore_axis_name="s")

    @functools.partial(
        pl.kernel, mesh=mesh,
        out_type=jax.ShapeDtypeStruct((F, N_PAD), jnp.float32),
        compiler_params=pltpu.CompilerParams(
            use_tc_tiling_on_sc=False, needs_layout_passes=False),
        scratch_types=[
            pltpu.VMEM((N_PAD,), jnp.float32),
            pltpu.VMEM((_SCH,), jnp.int32),
            pltpu.VMEM((_SCH,), jnp.float32),
            pltpu.SemaphoreType.DMA,
        ],
    )
    def k(mt_hbm, dst_hbm, out_hbm, acc_v, di_v, val_v, sem):
        fid = lax.axis_index("s") * _NC + lax.axis_index("c")
        zero16 = jnp.zeros((16,), jnp.float32)

        def zbody(i, c):
            acc_v[pl.ds(i * 16, 16)] = zero16
            return c

        lax.fori_loop(0, N_PAD // 16, zbody, 0)

        def chunk(ci, c):
            off = ci * _SCH
            pltpu.sync_copy(dst_hbm.at[pl.ds(off, _SCH)], di_v)
            pltpu.sync_copy(mt_hbm.at[fid, pl.ds(off, _SCH)], val_v)

            def vbody(kk, flag):
                sl = pl.ds(kk * 16, 16)
                d = di_v[sl]
                v = val_v[sl]
                cur = plsc.load_gather(acc_v, [d])
                plsc.store_scatter(acc_v, [d], jnp.maximum(v, cur))
                cur2 = plsc.load_gather(acc_v, [d])
                return flag | (v > cur2)

            flag = lax.fori_loop(0, _SCH // 16, vbody,
                                 jnp.zeros((16,), jnp.bool_))

            def fix_cond(pend):
                return pend > 0

            def fix_body(pend):
                def fbody(kk, flag2):
                    sl = pl.ds(kk * 16, 16)
                    d = di_v[sl]
                    v = val_v[sl]
                    cur = plsc.load_gather(acc_v, [d])
                    need = v > cur
                    plsc.store_scatter(acc_v, [d], v, mask=need)
                    cur2 = plsc.load_gather(acc_v, [d])
                    return flag2 | (v > cur2)

                flag2 = lax.fori_loop(0, _SCH // 16, fbody,
                                      jnp.zeros((16,), jnp.bool_))
                return jnp.max(flag2.astype(jnp.int32))

            lax.while_loop(fix_cond, fix_body,
                           jnp.max(flag.astype(jnp.int32)))
            return c

        lax.fori_loop(0, _SNCH, chunk, 0)
        pltpu.sync_copy(acc_v, out_hbm.at[fid])

    return k(mt, dst)


def _pool_classify_body(h_ref, wc_ref, bc_ref, out_ref):
    out_ref[...] = jax.lax.dot_general(
        h_ref[...], wc_ref[...], (((1,), (0,)), ((), ())),
        preferred_element_type=jnp.float32) + bc_ref[...]


def kernel(pos, edge_index, batch, W1a, b1a, W1b, b1b, W2a, b2a, W2b, b2b, Wc, bc):
    src = edge_index[0]
    dst = edge_index[1]
    A, B = _node_tables(pos, pos, W1a[:3], W1a[3:], b1a)
    e1 = _gather_diff_relu(A, B, src, dst)
    h1t = _scatter_max(_edge_mlp_t(e1, W1b, b1b), dst)
    pos_pad = jnp.pad(pos, ((0, N_PAD - N_NODES), (0, 0)))
    A2, B2 = _node_tables_t(h1t, pos_pad, W2a[:F], W2a[F:], b2a)
    e2 = _gather_diff_relu(A2, B2, src, dst)
    h2t = _scatter_max(_edge_mlp_t(e2, W2b, b2b), dst)
    pooled = jnp.maximum(
        jax.ops.segment_max(h2t[:, :N_NODES].T, batch, num_segments=64), 0.0)
    nc = Wc.shape[1]
    out = pl.pallas_call(
        _pool_classify_body,
        in_specs=[
            pl.BlockSpec((64, F), lambda: (0, 0)),
            pl.BlockSpec((F, nc), lambda: (0, 0)),
            pl.BlockSpec((1, nc), lambda: (0, 0)),
        ],
        out_specs=pl.BlockSpec((64, nc), lambda: (0, 0)),
        out_shape=jax.ShapeDtypeStruct((64, nc), jnp.float32),
    )(pooled, Wc, bc[None, :])
    return out


# speedup vs baseline: 1.0498x; 1.0498x over previous
import functools

import jax
import jax.numpy as jnp
from jax import lax
from jax.experimental import pallas as pl
from jax.experimental.pallas import tpu as pltpu
from jax.experimental.pallas import tpu_sc as plsc

N_NODES = 100000
N_PAD = 102400
N_EDGES = 1600000
F = 32

_NODE_BLK = 2000
_EDGE_BLK = 12800

_NC = 2
_NS = 16
_NW = _NC * _NS
_EPT = N_EDGES // _NW
_GCH = 1000
_GNCH = _EPT // _GCH


def _gather_diff_relu(A, B, src, dst):
    mesh = plsc.VectorSubcoreMesh(core_axis_name="c", subcore_axis_name="s")

    @functools.partial(
        pl.kernel, mesh=mesh,
        out_type=jax.ShapeDtypeStruct((N_EDGES, F), jnp.float32),
        compiler_params=pltpu.CompilerParams(use_tc_tiling_on_sc=False),
        scratch_types=[
            pltpu.VMEM((_GCH,), jnp.int32),
            pltpu.VMEM((_GCH,), jnp.int32),
            pltpu.VMEM((_GCH, F), jnp.float32),
            pltpu.VMEM((_GCH, F), jnp.float32),
            pltpu.SemaphoreType.DMA,
            pltpu.SemaphoreType.DMA,
        ],
    )
    def k(a_hbm, b_hbm, src_hbm, dst_hbm, out_hbm, si_v, di_v, ar_v, br_v, sem,
          sem2):
        wid = lax.axis_index("s") * _NC + lax.axis_index("c")
        base = wid * _EPT

        def chunk_body(ci, carry):
            off = base + ci * _GCH
            c1 = pltpu.async_copy(src_hbm.at[pl.ds(off, _GCH)], si_v, sem)
            c2 = pltpu.async_copy(dst_hbm.at[pl.ds(off, _GCH)], di_v, sem2)
            c1.wait()
            c2.wait()
            c3 = pltpu.async_copy(a_hbm.at[si_v], ar_v, sem)
            c4 = pltpu.async_copy(b_hbm.at[di_v], br_v, sem2)
            c3.wait()
            c4.wait()

            @plsc.parallel_loop(0, _GCH, unroll=8)
            def row_body(i):
                for j in range(F // 16):
                    sl = pl.ds(j * 16, 16)
                    ar_v[i, sl] = jnp.maximum(ar_v[i, sl] - br_v[i, sl], 0.0)

            pltpu.sync_copy(ar_v, out_hbm.at[pl.ds(off, _GCH)])
            return carry

        lax.fori_loop(0, _GNCH, chunk_body, 0)

    return k(A, B, src, dst)


def _node_tables_body(h_ref, pos_ref, wh_ref, wp_ref, b_ref, a_ref, b_out_ref):
    pq = jax.lax.dot_general(pos_ref[...], wp_ref[...], (((1,), (0,)), ((), ())),
                             preferred_element_type=jnp.float32)
    hq = jax.lax.dot_general(h_ref[...], wh_ref[...], (((1,), (0,)), ((), ())),
                             preferred_element_type=jnp.float32)
    a_ref[...] = hq + pq + b_ref[...]
    b_out_ref[...] = pq


def _node_tables(h, pos, Wh, Wp, b):
    n = h.shape[0]
    k = h.shape[1]
    grid = n // _NODE_BLK
    return pl.pallas_call(
        _node_tables_body,
        grid=(grid,),
        in_specs=[
            pl.BlockSpec((_NODE_BLK, k), lambda i: (i, 0)),
            pl.BlockSpec((_NODE_BLK, 3), lambda i: (i, 0)),
            pl.BlockSpec((k, F), lambda i: (0, 0)),
            pl.BlockSpec((3, F), lambda i: (0, 0)),
            pl.BlockSpec((1, F), lambda i: (0, 0)),
        ],
        out_specs=[
            pl.BlockSpec((_NODE_BLK, F), lambda i: (i, 0)),
            pl.BlockSpec((_NODE_BLK, F), lambda i: (i, 0)),
        ],
        out_shape=[
            jax.ShapeDtypeStruct((n, F), jnp.float32),
            jax.ShapeDtypeStruct((n, F), jnp.float32),
        ],
    )(h, pos, Wh, Wp, b[None, :])


def _edge_mlp_t_body(e_ref, wb_ref, bb_ref, out_ref):
    out_ref[...] = jax.lax.dot_general(
        wb_ref[...], e_ref[...], (((0,), (1,)), ((), ())),
        preferred_element_type=jnp.float32) + bb_ref[...]


def _edge_mlp_t(e, Wb, bb):
    ne = e.shape[0]
    grid = ne // _EDGE_BLK
    return pl.pallas_call(
        _edge_mlp_t_body,
        grid=(grid,),
        in_specs=[
            pl.BlockSpec((_EDGE_BLK, F), lambda i: (i, 0)),
            pl.BlockSpec((F, F), lambda i: (0, 0)),
            pl.BlockSpec((F, 1), lambda i: (0, 0)),
        ],
        out_specs=pl.BlockSpec((F, _EDGE_BLK), lambda i: (0, i)),
        out_shape=jax.ShapeDtypeStruct((F, ne), jnp.float32),
    )(e, Wb, bb[:, None])


def _node_tables_t_body(ht_ref, pos_ref, wh_ref, wp_ref, b_ref, a_ref, b_out_ref):
    pq = jax.lax.dot_general(pos_ref[...], wp_ref[...], (((1,), (0,)), ((), ())),
                             preferred_element_type=jnp.float32)
    hq = jax.lax.dot_general(ht_ref[...], wh_ref[...], (((0,), (0,)), ((), ())),
                             preferred_element_type=jnp.float32)
    a_ref[...] = hq + pq + b_ref[...]
    b_out_ref[...] = pq


def _node_tables_t(ht, pos, Wh, Wp, b):
    n = pos.shape[0]
    blk = 12800
    grid = n // blk
    return pl.pallas_call(
        _node_tables_t_body,
        grid=(grid,),
        in_specs=[
            pl.BlockSpec((F, blk), lambda i: (0, i)),
            pl.BlockSpec((blk, 3), lambda i: (i, 0)),
            pl.BlockSpec((F, F), lambda i: (0, 0)),
            pl.BlockSpec((3, F), lambda i: (0, 0)),
            pl.BlockSpec((1, F), lambda i: (0, 0)),
        ],
        out_specs=[
            pl.BlockSpec((blk, F), lambda i: (i, 0)),
            pl.BlockSpec((blk, F), lambda i: (i, 0)),
        ],
        out_shape=[
            jax.ShapeDtypeStruct((n, F), jnp.float32),
            jax.ShapeDtypeStruct((n, F), jnp.float32),
        ],
    )(ht, pos, Wh, Wp, b[None, :])


_SCH = 8000
_SNCH = N_EDGES // _SCH


def _scatter_max(mt, dst):
    mesh = plsc.VectorSubcoreMesh(core_axis_name="c", subcore_axis_name="s")

    @functools.partial(
        pl.kernel, mesh=mesh,
        out_type=jax.ShapeDtypeStruct((F, N_PAD), jnp.float32),
        compiler_params=pltpu.CompilerParams(
            use_tc_tiling_on_sc=False, needs_layout_passes=False),
        scratch_types=[
            pltpu.VMEM((N_PAD,), jnp.float32),
            pltpu.VMEM((_SCH,), jnp.int32),
            pltpu.VMEM((_SCH,), jnp.float32),
            pltpu.SemaphoreType.DMA,
            pltpu.SemaphoreType.DMA,
        ],
    )
    def k(mt_hbm, dst_hbm, out_hbm, acc_v, di_v, val_v, sem, sem2):
        fid = lax.axis_index("s") * _NC + lax.axis_index("c")
        zero16 = jnp.zeros((16,), jnp.float32)

        @plsc.parallel_loop(0, N_PAD // 16, unroll=8)
        def zbody(i):
            acc_v[pl.ds(i * 16, 16)] = zero16

        def chunk(ci, c):
            off = ci * _SCH
            c1 = pltpu.async_copy(dst_hbm.at[pl.ds(off, _SCH)], di_v, sem)
            c2 = pltpu.async_copy(mt_hbm.at[fid, pl.ds(off, _SCH)], val_v, sem2)
            c1.wait()
            c2.wait()

            def vbody(kk, flag):
                sl = pl.ds(kk * 16, 16)
                d = di_v[sl]
                v = val_v[sl]
                cur = plsc.load_gather(acc_v, [d])
                plsc.store_scatter(acc_v, [d], jnp.maximum(v, cur))
                cur2 = plsc.load_gather(acc_v, [d])
                return flag | (v > cur2)

            flag = lax.fori_loop(0, _SCH // 16, vbody,
                                 jnp.zeros((16,), jnp.bool_), unroll=8)

            def fix_cond(pend):
                return pend > 0

            def fix_body(pend):
                def fbody(kk, flag2):
                    sl = pl.ds(kk * 16, 16)
                    d = di_v[sl]
                    v = val_v[sl]
                    cur = plsc.load_gather(acc_v, [d])
                    need = v > cur
                    plsc.store_scatter(acc_v, [d], v, mask=need)
                    cur2 = plsc.load_gather(acc_v, [d])
                    return flag2 | (v > cur2)

                flag2 = lax.fori_loop(0, _SCH // 16, fbody,
                                      jnp.zeros((16,), jnp.bool_))
                return jnp.max(flag2.astype(jnp.int32))

            lax.while_loop(fix_cond, fix_body,
                           jnp.max(flag.astype(jnp.int32)))
            return c

        lax.fori_loop(0, _SNCH, chunk, 0)
        pltpu.sync_copy(acc_v, out_hbm.at[fid])

    return k(mt, dst)


def _pool_classify_body(h_ref, wc_ref, bc_ref, out_ref):
    out_ref[...] = jax.lax.dot_general(
        h_ref[...], wc_ref[...], (((1,), (0,)), ((), ())),
        preferred_element_type=jnp.float32) + bc_ref[...]


def kernel(pos, edge_index, batch, W1a, b1a, W1b, b1b, W2a, b2a, W2b, b2b, Wc, bc):
    src = edge_index[0]
    dst = edge_index[1]
    A, B = _node_tables(pos, pos, W1a[:3], W1a[3:], b1a)
    e1 = _gather_diff_relu(A, B, src, dst)
    h1t = _scatter_max(_edge_mlp_t(e1, W1b, b1b), dst)
    pos_pad = jnp.pad(pos, ((0, N_PAD - N_NODES), (0, 0)))
    A2, B2 = _node_tables_t(h1t, pos_pad, W2a[:F], W2a[F:], b2a)
    e2 = _gather_diff_relu(A2, B2, src, dst)
    h2t = _scatter_max(_edge_mlp_t(e2, W2b, b2b), dst)
    pooled = jnp.maximum(
        jax.ops.segment_max(h2t[:, :N_NODES].T, batch, num_segments=64), 0.0)
    nc = Wc.shape[1]
    out = pl.pallas_call(
        _pool_classify_body,
        in_specs=[
            pl.BlockSpec((64, F), lambda: (0, 0)),
            pl.BlockSpec((F, nc), lambda: (0, 0)),
            pl.BlockSpec((1, nc), lambda: (0, 0)),
        ],
        out_specs=pl.BlockSpec((64, nc), lambda: (0, 0)),
        out_shape=jax.ShapeDtypeStruct((64, nc), jnp.float32),
    )(pooled, Wc, bc[None, :])
    return out


# speedup vs baseline: 1.3611x; 1.2966x over previous
import functools

import jax
import jax.numpy as jnp
from jax import lax
from jax.experimental import pallas as pl
from jax.experimental.pallas import tpu as pltpu
from jax.experimental.pallas import tpu_sc as plsc

N_NODES = 100000
N_EDGES = 1600000
F = 32

_NODE_BLK = 2000
_EDGE_BLK = 12800

_NC = 2
_NS = 16
_NW = _NC * _NS
_EPT = N_EDGES // _NW
_GCH = 1000
_GNCH = _EPT // _GCH


def _gather_diff_relu(A, B, src, dst):
    mesh = plsc.VectorSubcoreMesh(core_axis_name="c", subcore_axis_name="s")

    @functools.partial(
        pl.kernel, mesh=mesh,
        out_type=jax.ShapeDtypeStruct((N_EDGES, F), jnp.float32),
        compiler_params=pltpu.CompilerParams(use_tc_tiling_on_sc=False),
        scratch_types=[
            pltpu.VMEM((_GCH,), jnp.int32),
            pltpu.VMEM((_GCH,), jnp.int32),
            pltpu.VMEM((_GCH, F), jnp.float32),
            pltpu.VMEM((_GCH, F), jnp.float32),
            pltpu.SemaphoreType.DMA,
            pltpu.SemaphoreType.DMA,
        ],
    )
    def k(a_hbm, b_hbm, src_hbm, dst_hbm, out_hbm, si_v, di_v, ar_v, br_v, sem,
          sem2):
        wid = lax.axis_index("s") * _NC + lax.axis_index("c")
        base = wid * _EPT

        def chunk_body(ci, carry):
            off = base + ci * _GCH
            c1 = pltpu.async_copy(src_hbm.at[pl.ds(off, _GCH)], si_v, sem)
            c2 = pltpu.async_copy(dst_hbm.at[pl.ds(off, _GCH)], di_v, sem2)
            c1.wait()
            c2.wait()
            c3 = pltpu.async_copy(a_hbm.at[si_v], ar_v, sem)
            c4 = pltpu.async_copy(b_hbm.at[di_v], br_v, sem2)
            c3.wait()
            c4.wait()

            @plsc.parallel_loop(0, _GCH, unroll=8)
            def row_body(i):
                for j in range(F // 16):
                    sl = pl.ds(j * 16, 16)
                    ar_v[i, sl] = jnp.maximum(ar_v[i, sl] - br_v[i, sl], 0.0)

            pltpu.sync_copy(ar_v, out_hbm.at[pl.ds(off, _GCH)])
            return carry

        lax.fori_loop(0, _GNCH, chunk_body, 0)

    return k(A, B, src, dst)


def _node_tables_body(h_ref, pos_ref, wh_ref, wp_ref, b_ref, a_ref, b_out_ref):
    pq = jax.lax.dot_general(pos_ref[...], wp_ref[...], (((1,), (0,)), ((), ())),
                             preferred_element_type=jnp.float32)
    hq = jax.lax.dot_general(h_ref[...], wh_ref[...], (((1,), (0,)), ((), ())),
                             preferred_element_type=jnp.float32)
    a_ref[...] = hq + pq + b_ref[...]
    b_out_ref[...] = pq


def _node_tables(h, pos, Wh, Wp, b):
    n = h.shape[0]
    k = h.shape[1]
    grid = n // _NODE_BLK
    return pl.pallas_call(
        _node_tables_body,
        grid=(grid,),
        in_specs=[
            pl.BlockSpec((_NODE_BLK, k), lambda i: (i, 0)),
            pl.BlockSpec((_NODE_BLK, 3), lambda i: (i, 0)),
            pl.BlockSpec((k, F), lambda i: (0, 0)),
            pl.BlockSpec((3, F), lambda i: (0, 0)),
            pl.BlockSpec((1, F), lambda i: (0, 0)),
        ],
        out_specs=[
            pl.BlockSpec((_NODE_BLK, F), lambda i: (i, 0)),
            pl.BlockSpec((_NODE_BLK, F), lambda i: (i, 0)),
        ],
        out_shape=[
            jax.ShapeDtypeStruct((n, F), jnp.float32),
            jax.ShapeDtypeStruct((n, F), jnp.float32),
        ],
    )(h, pos, Wh, Wp, b[None, :])


def _edge_mlp_body(e_ref, wb_ref, bb_ref, out_ref):
    out_ref[...] = jax.lax.dot_general(
        e_ref[...], wb_ref[...], (((1,), (0,)), ((), ())),
        preferred_element_type=jnp.float32) + bb_ref[...]


def _edge_mlp(e, Wb, bb):
    ne = e.shape[0]
    grid = ne // _EDGE_BLK
    return pl.pallas_call(
        _edge_mlp_body,
        grid=(grid,),
        in_specs=[
            pl.BlockSpec((_EDGE_BLK, F), lambda i: (i, 0)),
            pl.BlockSpec((F, F), lambda i: (0, 0)),
            pl.BlockSpec((1, F), lambda i: (0, 0)),
        ],
        out_specs=pl.BlockSpec((_EDGE_BLK, F), lambda i: (i, 0)),
        out_shape=jax.ShapeDtypeStruct((ne, F), jnp.float32),
    )(e, Wb, bb[None, :])


def _layer(h, pos, src, dst, Wh, Wp, ba, Wb, bb):
    A, B = _node_tables(h, pos, Wh, Wp, ba)
    e = _gather_diff_relu(A, B, src, dst)
    msg = _edge_mlp(e, Wb, bb)
    agg = jax.ops.segment_max(msg, dst, num_segments=N_NODES)
    return jnp.maximum(agg, 0.0)


def _pool_classify_body(h_ref, wc_ref, bc_ref, out_ref):
    out_ref[...] = jax.lax.dot_general(
        h_ref[...], wc_ref[...], (((1,), (0,)), ((), ())),
        preferred_element_type=jnp.float32) + bc_ref[...]


def kernel(pos, edge_index, batch, W1a, b1a, W1b, b1b, W2a, b2a, W2b, b2b, Wc, bc):
    src = edge_index[0]
    dst = edge_index[1]
    h = _layer(pos, pos, src, dst, W1a[:3], W1a[3:], b1a, W1b, b1b)
    h = _layer(h, pos, src, dst, W2a[:F], W2a[F:], b2a, W2b, b2b)
    pooled = jnp.maximum(jax.ops.segment_max(h, batch, num_segments=64), 0.0)
    nc = Wc.shape[1]
    out = pl.pallas_call(
        _pool_classify_body,
        in_specs=[
            pl.BlockSpec((64, F), lambda: (0, 0)),
            pl.BlockSpec((F, nc), lambda: (0, 0)),
            pl.BlockSpec((1, nc), lambda: (0, 0)),
        ],
        out_specs=pl.BlockSpec((64, nc), lambda: (0, 0)),
        out_shape=jax.ShapeDtypeStruct((64, nc), jnp.float32),
    )(pooled, Wc, bc[None, :])
    return out


# speedup vs baseline: 1.4478x; 1.0637x over previous
import functools

import jax
import jax.numpy as jnp
from jax import lax
from jax.experimental import pallas as pl
from jax.experimental.pallas import tpu as pltpu
from jax.experimental.pallas import tpu_sc as plsc

N_NODES = 100000
N_EDGES = 1600000
F = 32

_NODE_BLK = 2000
_EDGE_BLK = 12800

_NC = 2
_NS = 16
_NW = _NC * _NS
_EPT = N_EDGES // _NW
_GCH = 1000
_GNCH = _EPT // _GCH


def _gather_diff_relu(A, B, src, dst):
    mesh = plsc.VectorSubcoreMesh(core_axis_name="c", subcore_axis_name="s")

    @functools.partial(
        pl.kernel, mesh=mesh,
        out_type=jax.ShapeDtypeStruct((N_EDGES // 4, 4 * F), jnp.float32),
        compiler_params=pltpu.CompilerParams(use_tc_tiling_on_sc=False),
        scratch_types=[
            pltpu.VMEM((_GCH,), jnp.int32),
            pltpu.VMEM((_GCH,), jnp.int32),
            pltpu.VMEM((_GCH, F), jnp.float32),
            pltpu.VMEM((_GCH, F), jnp.float32),
            pltpu.VMEM((_GCH // 4, 4 * F), jnp.float32),
            pltpu.SemaphoreType.DMA,
            pltpu.SemaphoreType.DMA,
        ],
    )
    def k(a_hbm, b_hbm, src_hbm, dst_hbm, out_hbm, si_v, di_v, ar_v, br_v,
          out_v, sem, sem2):
        wid = lax.axis_index("s") * _NC + lax.axis_index("c")
        base = wid * _EPT

        def chunk_body(ci, carry):
            off = base + ci * _GCH
            c1 = pltpu.async_copy(src_hbm.at[pl.ds(off, _GCH)], si_v, sem)
            c2 = pltpu.async_copy(dst_hbm.at[pl.ds(off, _GCH)], di_v, sem2)
            c1.wait()
            c2.wait()
            c3 = pltpu.async_copy(a_hbm.at[si_v], ar_v, sem)
            c4 = pltpu.async_copy(b_hbm.at[di_v], br_v, sem2)
            c3.wait()
            c4.wait()

            @plsc.parallel_loop(0, _GCH // 4, unroll=4)
            def row_body(i):
                for q in range(4):
                    for j in range(F // 16):
                        sl = pl.ds(j * 16, 16)
                        osl = pl.ds(q * F + j * 16, 16)
                        out_v[i, osl] = jnp.maximum(
                            ar_v[i * 4 + q, sl] - br_v[i * 4 + q, sl], 0.0)

            pltpu.sync_copy(out_v, out_hbm.at[pl.ds(off // 4, _GCH // 4)])
            return carry

        lax.fori_loop(0, _GNCH, chunk_body, 0)

    return k(A, B, src, dst)


def _node_tables_body(h_ref, pos_ref, wh_ref, wp_ref, b_ref, a_ref, b_out_ref):
    pq = jax.lax.dot_general(pos_ref[...], wp_ref[...], (((1,), (0,)), ((), ())),
                             preferred_element_type=jnp.float32)
    hq = jax.lax.dot_general(h_ref[...], wh_ref[...], (((1,), (0,)), ((), ())),
                             preferred_element_type=jnp.float32)
    a_ref[...] = hq + pq + b_ref[...]
    b_out_ref[...] = pq


def _node_tables(h, pos, Wh, Wp, b):
    n = h.shape[0]
    k = h.shape[1]
    grid = n // _NODE_BLK
    return pl.pallas_call(
        _node_tables_body,
        grid=(grid,),
        in_specs=[
            pl.BlockSpec((_NODE_BLK, k), lambda i: (i, 0)),
            pl.BlockSpec((_NODE_BLK, 3), lambda i: (i, 0)),
            pl.BlockSpec((k, F), lambda i: (0, 0)),
            pl.BlockSpec((3, F), lambda i: (0, 0)),
            pl.BlockSpec((1, F), lambda i: (0, 0)),
        ],
        out_specs=[
            pl.BlockSpec((_NODE_BLK, F), lambda i: (i, 0)),
            pl.BlockSpec((_NODE_BLK, F), lambda i: (i, 0)),
        ],
        out_shape=[
            jax.ShapeDtypeStruct((n, F), jnp.float32),
            jax.ShapeDtypeStruct((n, F), jnp.float32),
        ],
    )(h, pos, Wh, Wp, b[None, :])


def _edge_mlp_body(e_ref, wb_ref, bb_ref, out_ref):
    out_ref[...] = jax.lax.dot_general(
        e_ref[...], wb_ref[...], (((1,), (0,)), ((), ())),
        preferred_element_type=jnp.float32) + bb_ref[...]


def _edge_mlp(e_fold, Wb, bb):
    nr = e_fold.shape[0]
    blk = _EDGE_BLK // 4
    grid = nr // blk
    w_block = jnp.kron(jnp.eye(4, dtype=jnp.float32), Wb)
    bb_tile = jnp.tile(bb, 4)[None, :]
    return pl.pallas_call(
        _edge_mlp_body,
        grid=(grid,),
        in_specs=[
            pl.BlockSpec((blk, 4 * F), lambda i: (i, 0)),
            pl.BlockSpec((4 * F, 4 * F), lambda i: (0, 0)),
            pl.BlockSpec((1, 4 * F), lambda i: (0, 0)),
        ],
        out_specs=pl.BlockSpec((blk, 4 * F), lambda i: (i, 0)),
        out_shape=jax.ShapeDtypeStruct((nr, 4 * F), jnp.float32),
    )(e_fold, w_block, bb_tile)


def _layer(h, pos, src, dst, Wh, Wp, ba, Wb, bb):
    A, B = _node_tables(h, pos, Wh, Wp, ba)
    e_fold = _gather_diff_relu(A, B, src, dst)
    msg = jnp.reshape(_edge_mlp(e_fold, Wb, bb), (N_EDGES, F))
    agg = jax.ops.segment_max(msg, dst, num_segments=N_NODES)
    return jnp.maximum(agg, 0.0)


def _pool_classify_body(h_ref, wc_ref, bc_ref, out_ref):
    out_ref[...] = jax.lax.dot_general(
        h_ref[...], wc_ref[...], (((1,), (0,)), ((), ())),
        preferred_element_type=jnp.float32) + bc_ref[...]


def kernel(pos, edge_index, batch, W1a, b1a, W1b, b1b, W2a, b2a, W2b, b2b, Wc, bc):
    src = edge_index[0]
    dst = edge_index[1]
    h = _layer(pos, pos, src, dst, W1a[:3], W1a[3:], b1a, W1b, b1b)
    h = _layer(h, pos, src, dst, W2a[:F], W2a[F:], b2a, W2b, b2b)
    pooled = jnp.maximum(jax.ops.segment_max(h, batch, num_segments=64), 0.0)
    nc = Wc.shape[1]
    out = pl.pallas_call(
        _pool_classify_body,
        in_specs=[
            pl.BlockSpec((64, F), lambda: (0, 0)),
            pl.BlockSpec((F, nc), lambda: (0, 0)),
            pl.BlockSpec((1, nc), lambda: (0, 0)),
        ],
        out_specs=pl.BlockSpec((64, nc), lambda: (0, 0)),
        out_shape=jax.ShapeDtypeStruct((64, nc), jnp.float32),
    )(pooled, Wc, bc[None, :])
    return out
